# baseline (device time: 22007 ns/iter reference)
import jax
import jax.numpy as jnp
from jax import lax
from jax.experimental import pallas as pl
from jax.experimental.pallas import tpu as pltpu

NC = 6


def kernel(A, B):
    m, k = A.shape
    _, n = B.shape
    half = m // 2
    rh = half // NC

    def body(a_ref, b_ref, out_ref, ah_ref, acc_ref, commx_ref, sx, rx, sy, ry):
        my_x = lax.axis_index("x")
        my_y = lax.axis_index("y")
        xpeer = (1 - my_x, my_y)
        ypeer = (my_x, 1 - my_y)

        barrier_sem = pltpu.get_barrier_semaphore()
        for nbr in (xpeer, ypeer):
            pl.semaphore_signal(
                barrier_sem, inc=1,
                device_id=nbr, device_id_type=pl.DeviceIdType.MESH,
            )
        pl.semaphore_wait(barrier_sem, 2)

        row0 = my_y * half

        ah_ref[...] = a_ref[pl.ds(row0, half), :]

        def rows(c):
            return pl.ds(c * rh, rh)

        def out_rows(c):
            return pl.ds(row0 + c * rh, rh)

        xd = [
            pltpu.make_async_remote_copy(
                src_ref=acc_ref.at[rows(c), :],
                dst_ref=commx_ref.at[rows(c), :],
                send_sem=sx.at[c],
                recv_sem=rx.at[c],
                device_id=xpeer,
                device_id_type=pl.DeviceIdType.MESH,
            )
            for c in range(NC)
        ]
        yd = [
            pltpu.make_async_remote_copy(
                src_ref=out_ref.at[out_rows(c), :],
                dst_ref=out_ref.at[out_rows(c), :],
                send_sem=sy.at[c],
                recv_sem=ry.at[c],
                device_id=ypeer,
                device_id_type=pl.DeviceIdType.MESH,
            )
            for c in range(NC)
        ]

        def compute(c):
            acc_ref[rows(c), :] = jnp.dot(
                ah_ref[rows(c), :], b_ref[...],
                preferred_element_type=jnp.float32,
            )

        def finish(c):
            out_ref[out_rows(c), :] = acc_ref[rows(c), :] + commx_ref[rows(c), :]
            yd[c].start()

        for c in range(NC):
            compute(c)
            xd[c].start()
        for c in range(NC):
            xd[c].wait()
            out_ref[out_rows(c), :] = acc_ref[rows(c), :] + commx_ref[rows(c), :]
            out_ref[pl.ds((1 - my_y) * half + c * rh, rh), :] = acc_ref[rows(c), :]

    return pl.pallas_call(
        body,
        out_shape=jax.ShapeDtypeStruct((m, n), jnp.float32),
        in_specs=[
            pl.BlockSpec(memory_space=pltpu.VMEM),
            pl.BlockSpec(memory_space=pltpu.VMEM),
        ],
        out_specs=pl.BlockSpec(memory_space=pltpu.VMEM),
        scratch_shapes=[
            pltpu.VMEM((half, k), jnp.float32),
            pltpu.VMEM((half, n), jnp.float32),
            pltpu.VMEM((half, n), jnp.float32),
            pltpu.SemaphoreType.DMA((NC,)),
            pltpu.SemaphoreType.DMA((NC,)),
            pltpu.SemaphoreType.DMA((NC,)),
            pltpu.SemaphoreType.DMA((NC,)),
        ],
        compiler_params=pltpu.CompilerParams(collective_id=0),
    )(A, B)


# device time: 17719 ns/iter; 1.2420x vs baseline; 1.2420x over previous
import os

import jax
import jax.numpy as jnp
from jax import lax
from jax.experimental import pallas as pl
from jax.experimental.pallas import tpu as pltpu

NC = int(os.environ.get("NC", "8"))


def kernel(A, B):
    m, k = A.shape
    _, n = B.shape
    half = m // 2
    rh = half // NC

    def body(a_ref, b_ref, out_ref, ah_ref, acc_ref,
             xsend_ref, xrecv_ref, ysend_ref, yrecv_ref,
             sx, rx, sy, ry):
        my_x = lax.axis_index("x")
        my_y = lax.axis_index("y")
        xpeer = (1 - my_x, my_y)
        ypeer = (my_x, 1 - my_y)

        barrier_sem = pltpu.get_barrier_semaphore()
        for nbr in (xpeer, ypeer):
            pl.semaphore_signal(
                barrier_sem, inc=1,
                device_id=nbr, device_id_type=pl.DeviceIdType.MESH,
            )
        pl.semaphore_wait(barrier_sem, 2)

        row0 = my_y * half

        ah_ref[...] = a_ref[pl.ds(row0, half), :]

        def rows(c):
            return pl.ds(c * rh, rh)

        xd = [
            pltpu.make_async_remote_copy(
                src_ref=xsend_ref.at[rows(c), :],
                dst_ref=xrecv_ref.at[rows(c), :],
                send_sem=sx.at[c],
                recv_sem=rx.at[c],
                device_id=xpeer,
                device_id_type=pl.DeviceIdType.MESH,
            )
            for c in range(NC)
        ]
        yd = [
            pltpu.make_async_remote_copy(
                src_ref=ysend_ref.at[rows(c), :],
                dst_ref=yrecv_ref.at[rows(c), :],
                send_sem=sy.at[c],
                recv_sem=ry.at[c],
                device_id=ypeer,
                device_id_type=pl.DeviceIdType.MESH,
            )
            for c in range(NC)
        ]

        def compute(c):
            p = jnp.dot(
                ah_ref[rows(c), :], b_ref[...],
                preferred_element_type=jnp.float32,
            )
            acc_ref[rows(c), :] = p
            xsend_ref[rows(c), :] = p.astype(jnp.bfloat16)

        def finish(c):
            s = acc_ref[rows(c), :] + xrecv_ref[rows(c), :].astype(jnp.float32)
            out_ref[pl.ds(row0 + c * rh, rh), :] = s
            ysend_ref[rows(c), :] = s.astype(jnp.bfloat16)
            yd[c].start()

        for c in range(NC):
            compute(c)
            xd[c].start()
        for c in range(NC):
            xd[c].wait()
            finish(c)
        for c in range(NC):
            yd[c].wait()
            out_ref[pl.ds((1 - my_y) * half + c * rh, rh), :] = (
                yrecv_ref[rows(c), :].astype(jnp.float32)
            )

    return pl.pallas_call(
        body,
        out_shape=jax.ShapeDtypeStruct((m, n), jnp.float32),
        in_specs=[
            pl.BlockSpec(memory_space=pltpu.VMEM),
            pl.BlockSpec(memory_space=pltpu.VMEM),
        ],
        out_specs=pl.BlockSpec(memory_space=pltpu.VMEM),
        scratch_shapes=[
            pltpu.VMEM((half, k), jnp.float32),
            pltpu.VMEM((half, n), jnp.float32),
            pltpu.VMEM((half, n), jnp.bfloat16),
            pltpu.VMEM((half, n), jnp.bfloat16),
            pltpu.VMEM((half, n), jnp.bfloat16),
            pltpu.VMEM((half, n), jnp.bfloat16),
            pltpu.SemaphoreType.DMA((NC,)),
            pltpu.SemaphoreType.DMA((NC,)),
            pltpu.SemaphoreType.DMA((NC,)),
            pltpu.SemaphoreType.DMA((NC,)),
        ],
        compiler_params=pltpu.CompilerParams(collective_id=0),
    )(A, B)
